# baseline (speedup 1.0000x reference)
import functools

import jax
import jax.numpy as jnp
from jax import lax
from jax.experimental import pallas as pl
from jax.experimental.pallas import tpu as pltpu
from jax.experimental.pallas import tpu_sc as plsc

_NUM_CORES = 2
_NUM_SUBCORES = 16
_NW = _NUM_CORES * _NUM_SUBCORES
_L = 16
_V = 100000
_VP = 100096
_ROWS_F = _VP // 8
_FULLW = 1024
_NFULL = 96
_TAILW = 1792
_CHUNK = 256


def _sc_fused(e_t, e_tail, idx):
  f = e_t.shape[0]
  fh = f // 2
  n, = idx.shape
  per_w = n // _NW
  n_chunks = per_w // _CHUNK

  mesh = plsc.VectorSubcoreMesh(core_axis_name="c", subcore_axis_name="s")

  @functools.partial(
      pl.kernel,
      out_type=[
          jax.ShapeDtypeStruct((n * 16 // 128, 128), jnp.float32),
          jax.ShapeDtypeStruct((f * _ROWS_F, 128), jnp.float32),
      ],
      mesh=mesh,
      scratch_types=[
          pltpu.VMEM((_L, _TAILW), jnp.float32),
          pltpu.VMEM((_L, _FULLW), jnp.float32),
          pltpu.VMEM((_TAILW // 8, 128), jnp.float32),
          pltpu.VMEM((_CHUNK,), jnp.int32),
          pltpu.VMEM((_CHUNK,), jnp.int32),
          pltpu.VMEM((_CHUNK,), jnp.int32),
          pltpu.VMEM((_CHUNK, 128), jnp.float32),
          pltpu.VMEM((_CHUNK // 8, 128), jnp.float32),
          pltpu.SemaphoreType.DMA,
          pltpu.SemaphoreType.DMA,
      ],
      compiler_params=pltpu.CompilerParams(use_tc_tiling_on_sc=True,
                                           needs_layout_passes=False),
  )
  def fused_kernel(et_hbm, etail_hbm, idx_hbm, out_hbm, tab_hbm,
                   ina_v, inb_v, line_v, idx_v, q_v, r_v, buf_v, outc_v,
                   sem_a, sem_b):
    c = lax.axis_index("c")
    s = lax.axis_index("s")
    lanes = lax.iota(jnp.int32, _L)

    def stage(src, dst, width, vcol0, fg, sem):
      a = pltpu.async_copy(
          src.at[fg, pl.ds(0, 8), pl.ds(vcol0, width)],
          dst.at[pl.ds(0, 8), pl.ds(0, width)], sem)
      b = pltpu.async_copy(
          src.at[fg, pl.ds(8, 8), pl.ds(vcol0, width)],
          dst.at[pl.ds(8, 8), pl.ds(0, width)], sem)
      return a, b

    def repack(src_v, width, fg, row0):
      def grp(g, carry):
        vals = plsc.load_gather(src_v, [lanes, jnp.full((_L,), g, jnp.int32)])
        plsc.store_scatter(
            line_v,
            [jnp.full((_L,), g >> 3, jnp.int32),
             jnp.bitwise_and(g, 7) * 16 + lanes], vals)
        return carry

      lax.fori_loop(0, width, grp, 0, unroll=4)
      pltpu.sync_copy(
          line_v.at[pl.ds(0, width // 8)],
          tab_hbm.at[pl.ds(pl.multiple_of(row0, 8), width // 8)])

    def field_body(fl, carry):
      fg = c * fh + fl
      frow = fg * _ROWS_F
      ha = stage(et_hbm, ina_v, _FULLW, (s + 0 * _NUM_SUBCORES) * _FULLW,
                 fg, sem_a)
      for k in range(_NFULL // _NUM_SUBCORES):
        task = s + k * _NUM_SUBCORES
        cur, nxt = (ina_v, inb_v) if k % 2 == 0 else (inb_v, ina_v)
        hn = None
        if k + 1 < _NFULL // _NUM_SUBCORES:
          hn = stage(et_hbm, nxt, _FULLW,
                     (s + (k + 1) * _NUM_SUBCORES) * _FULLW, fg,
                     sem_b if k % 2 == 0 else sem_a)
        ha[0].wait()
        ha[1].wait()
        repack(cur, _FULLW, fg, frow + task * (_FULLW // 8))
        if hn is not None:
          ha = hn
      return carry

    lax.fori_loop(0, fh, field_body, 0)

    @pl.when(s < fh)
    def _tail():
      fg = c * fh + s
      ta, tb = stage(etail_hbm, ina_v, _TAILW, 0, fg, sem_a)
      ta.wait()
      tb.wait()
      repack(ina_v, _TAILW, fg,
             fg * _ROWS_F + _NFULL * (_FULLW // 8))

    plsc.subcore_barrier()

    wid = c * _NUM_SUBCORES + s
    base = pl.multiple_of(wid * per_w, per_w)

    def chunk_body(ch, carry):
      off = base + ch * _CHUNK
      pltpu.sync_copy(idx_hbm.at[pl.ds(off, _CHUNK)], idx_v)

      def split_body(i, carry2):
        ix = idx_v[pl.ds(i * _L, _L)]
        q_v[pl.ds(i * _L, _L)] = lax.shift_right_logical(ix, 3)
        r_v[pl.ds(i * _L, _L)] = lax.shift_left(jnp.bitwise_and(ix, 7), 4)
        return carry2

      lax.fori_loop(0, _CHUNK // _L, split_body, 0)
      pltpu.async_copy(tab_hbm.at[q_v], buf_v, sem_a).wait()

      def group_body(g, carry2):
        i0 = g * _L
        ivec = lanes + i0
        rvec = r_v[pl.ds(i0, _L)]
        orow = lax.shift_right_logical(ivec, 3)
        ocol = lax.shift_left(jnp.bitwise_and(ivec, 7), 4)
        for w in range(16):
          vals = plsc.load_gather(buf_v, [ivec, rvec + w])
          plsc.store_scatter(outc_v, [orow, ocol + w], vals)
        return carry2

      lax.fori_loop(0, _CHUNK // _L, group_body, 0)
      pltpu.sync_copy(
          outc_v,
          out_hbm.at[pl.ds(pl.multiple_of((off * 16) // 128, _CHUNK // 8),
                           _CHUNK // 8)])
      return carry

    lax.fori_loop(0, n_chunks, chunk_body, 0)

  return fused_kernel(e_t, e_tail, idx)


def _tc_mlp(x_num, emb3, w1n, w1c, b1, w2, b2, w3, b3, wab, bab, bm):
  b, nd = x_num.shape
  grid = (b // bm,)

  def body(xn_ref, emb_ref, w1n_ref, w1c_ref, b1_ref, w2_ref, b2_ref,
           w3_ref, b3_ref, wab_ref, bab_ref, out_ref):
    h = jnp.dot(xn_ref[...], w1n_ref[...], preferred_element_type=jnp.float32)
    for g in range(4):
      h = h + jnp.dot(emb_ref[g], w1c_ref[g],
                      preferred_element_type=jnp.float32)
    h = jnp.maximum(h + b1_ref[...], 0.0)
    h = jnp.maximum(
        jnp.dot(h, w2_ref[...], preferred_element_type=jnp.float32)
        + b2_ref[...], 0.0)
    h = jnp.maximum(
        jnp.dot(h, w3_ref[...], preferred_element_type=jnp.float32)
        + b3_ref[...], 0.0)
    out_ref[...] = (
        jnp.dot(h, wab_ref[...], preferred_element_type=jnp.float32)
        + bab_ref[...])

  full2 = lambda shape: pl.BlockSpec(shape, lambda i: (0, 0))
  full3 = lambda shape: pl.BlockSpec(shape, lambda i: (0, 0, 0))
  return pl.pallas_call(
      body,
      grid=grid,
      in_specs=[
          pl.BlockSpec((bm, nd), lambda i: (i, 0)),
          pl.BlockSpec((4, bm, 128), lambda i: (0, i, 0)),
          full2(w1n.shape),
          full3(w1c.shape),
          full2(b1.shape),
          full2(w2.shape),
          full2(b2.shape),
          full2(w3.shape),
          full2(b3.shape),
          full2(wab.shape),
          full2(bab.shape),
      ],
      out_specs=pl.BlockSpec((bm, 2), lambda i: (i, 0)),
      out_shape=jax.ShapeDtypeStruct((b, 2), jnp.float32),
  )(x_num, emb3, w1n, w1c, b1, w2, b2, w3, b3, wab, bab)


_PLANE_FIELDS = (list(range(0, 8)),
                 [8, 9, 10, 11, 12, 0, 1, 2],
                 list(range(13, 21)),
                 [21, 22, 23, 24, 25, 13, 14, 15])
_REAL_SLOTS = (8, 5, 8, 5)


def kernel(x_num, x_cat, E, W1, b1, W2, b2, W3, b3, WA, bA, WB, bB):
  f, v, d = E.shape
  b = x_cat.shape[0]
  nd = x_num.shape[1]

  e_t = jnp.transpose(E, (0, 2, 1))
  e_tail = jnp.pad(e_t[:, :, _NFULL * _FULLW:],
                   ((0, 0), (0, 0), (0, _TAILW - (v - _NFULL * _FULLW))))

  idx_all = x_cat + (jnp.arange(f, dtype=jnp.int32) * _VP)[None, :]
  cols = jnp.asarray(sum(_PLANE_FIELDS, []), dtype=jnp.int32)
  idx3 = jnp.take(idx_all, cols, axis=1).reshape(b, 4, 8)
  idx3 = idx3.transpose(1, 0, 2).reshape(-1)

  emb, _ = _sc_fused(e_t, e_tail, idx3)
  emb3 = emb.reshape(4, b, 8 * d)

  w1e = W1[nd:]
  blocks = []
  r0 = 0
  for p in range(4):
    nreal = _REAL_SLOTS[p] * d
    blk = w1e[r0:r0 + nreal]
    r0 += nreal
    if nreal < 128:
      blk = jnp.pad(blk, ((0, 128 - nreal), (0, 0)))
    blocks.append(blk)
  w1c = jnp.stack(blocks)

  wab = jnp.concatenate([WA, WB], axis=1)
  bab = jnp.concatenate([bA, bB])[None, :]
  out = _tc_mlp(x_num, emb3, W1[:nd], w1c, b1[None, :], W2, b2[None, :],
                W3, b3[None, :], wab, bab, bm=2048)
  return out[:, 0], out[:, 1]


# speedup vs baseline: 1.1839x; 1.1839x over previous
import functools

import jax
import jax.numpy as jnp
from jax import lax
from jax.experimental import pallas as pl
from jax.experimental.pallas import tpu as pltpu
from jax.experimental.pallas import tpu_sc as plsc

_NUM_CORES = 2
_NUM_SUBCORES = 16
_NW = _NUM_CORES * _NUM_SUBCORES
_L = 16
_V = 100000
_VP = 100096
_ROWS_F = _VP // 8
_FULLW = 1024
_NFULL = 96
_TAILW = 1792
_CHUNK = 256


def _sc_fused(e_t, e_tail, idx):
  f = e_t.shape[0]
  fh = f // 2
  n, = idx.shape
  per_w = n // _NW
  n_chunks = per_w // _CHUNK

  mesh = plsc.VectorSubcoreMesh(core_axis_name="c", subcore_axis_name="s")

  @functools.partial(
      pl.kernel,
      out_type=[
          jax.ShapeDtypeStruct((n * 16 // 128, 128), jnp.float32),
          jax.ShapeDtypeStruct((f * _ROWS_F, 128), jnp.float32),
      ],
      mesh=mesh,
      scratch_types=[
          pltpu.VMEM((_L, _TAILW), jnp.float32),
          pltpu.VMEM((_L, _FULLW), jnp.float32),
          pltpu.VMEM((_TAILW // 8, 128), jnp.float32),
          pltpu.VMEM((_CHUNK,), jnp.int32),
          pltpu.VMEM((_CHUNK,), jnp.int32),
          pltpu.VMEM((_CHUNK,), jnp.int32),
          pltpu.VMEM((_CHUNK, 128), jnp.float32),
          pltpu.VMEM((_CHUNK // 8, 128), jnp.float32),
          pltpu.SemaphoreType.DMA,
          pltpu.SemaphoreType.DMA,
      ],
      compiler_params=pltpu.CompilerParams(use_tc_tiling_on_sc=True,
                                           needs_layout_passes=False),
  )
  def fused_kernel(et_hbm, etail_hbm, idx_hbm, out_hbm, tab_hbm,
                   ina_v, inb_v, line_v, idx_v, q_v, r_v, buf_v, outc_v,
                   sem_a, sem_b):
    c = lax.axis_index("c")
    s = lax.axis_index("s")
    lanes = lax.iota(jnp.int32, _L)

    def stage(src, dst, width, vcol0, fg, sem):
      a = pltpu.async_copy(
          src.at[fg, pl.ds(0, 8), pl.ds(vcol0, width)],
          dst.at[pl.ds(0, 8), pl.ds(0, width)], sem)
      b = pltpu.async_copy(
          src.at[fg, pl.ds(8, 8), pl.ds(vcol0, width)],
          dst.at[pl.ds(8, 8), pl.ds(0, width)], sem)
      return a, b

    def repack(src_v, width, row0):
      def grp(gp, carry):
        g0 = gp * 2
        g1 = g0 + 1
        v0 = plsc.load_gather(src_v, [lanes, jnp.full((_L,), g0, jnp.int32)])
        v1 = plsc.load_gather(src_v, [lanes, jnp.full((_L,), g1, jnp.int32)])
        line_v[g0 >> 3, pl.ds(jnp.bitwise_and(g0, 7) * 16, _L)] = v0
        line_v[g1 >> 3, pl.ds(jnp.bitwise_and(g1, 7) * 16, _L)] = v1
        return carry

      lax.fori_loop(0, width // 2, grp, 0, unroll=4)
      pltpu.sync_copy(
          line_v.at[pl.ds(0, width // 8)],
          tab_hbm.at[pl.ds(pl.multiple_of(row0, 8), width // 8)])

    def field_body(fl, carry):
      fg = c * fh + fl
      frow = fg * _ROWS_F
      ha = stage(et_hbm, ina_v, _FULLW, (s + 0 * _NUM_SUBCORES) * _FULLW,
                 fg, sem_a)
      for k in range(_NFULL // _NUM_SUBCORES):
        task = s + k * _NUM_SUBCORES
        cur, nxt = (ina_v, inb_v) if k % 2 == 0 else (inb_v, ina_v)
        hn = None
        if k + 1 < _NFULL // _NUM_SUBCORES:
          hn = stage(et_hbm, nxt, _FULLW,
                     (s + (k + 1) * _NUM_SUBCORES) * _FULLW, fg,
                     sem_b if k % 2 == 0 else sem_a)
        ha[0].wait()
        ha[1].wait()
        repack(cur, _FULLW, frow + task * (_FULLW // 8))
        if hn is not None:
          ha = hn
      return carry

    lax.fori_loop(0, fh, field_body, 0)

    @pl.when(s < fh)
    def _tail():
      fg = c * fh + s
      ta, tb = stage(etail_hbm, ina_v, _TAILW, 0, fg, sem_a)
      ta.wait()
      tb.wait()
      repack(ina_v, _TAILW, fg * _ROWS_F + _NFULL * (_FULLW // 8))

    plsc.subcore_barrier()

    wid = c * _NUM_SUBCORES + s
    base = pl.multiple_of(wid * per_w, per_w)

    def chunk_body(ch, carry):
      off = base + ch * _CHUNK
      pltpu.sync_copy(idx_hbm.at[pl.ds(off, _CHUNK)], idx_v)

      def split_body(i, carry2):
        ix = idx_v[pl.ds(i * _L, _L)]
        q_v[pl.ds(i * _L, _L)] = lax.shift_right_logical(ix, 3)
        r_v[pl.ds(i * _L, _L)] = lax.shift_left(jnp.bitwise_and(ix, 7), 4)
        return carry2

      lax.fori_loop(0, _CHUNK // _L, split_body, 0)
      pltpu.async_copy(tab_hbm.at[q_v], buf_v, sem_a).wait()

      def group_body(g, carry2):
        i0 = g * _L
        ivec = lanes + i0
        rvec = r_v[pl.ds(i0, _L)]
        orow = lax.shift_right_logical(ivec, 3)
        ocol = lax.shift_left(jnp.bitwise_and(ivec, 7), 4)
        for w in range(16):
          vals = plsc.load_gather(buf_v, [ivec, rvec + w])
          plsc.store_scatter(outc_v, [orow, ocol + w], vals)
        return carry2

      lax.fori_loop(0, _CHUNK // _L, group_body, 0)
      pltpu.sync_copy(
          outc_v,
          out_hbm.at[pl.ds(pl.multiple_of((off * 16) // 128, _CHUNK // 8),
                           _CHUNK // 8)])
      return carry

    lax.fori_loop(0, n_chunks, chunk_body, 0)

  return fused_kernel(e_t, e_tail, idx)


def _tc_mlp(x_num, emb3, w1n, w1c, b1, w2, b2, w3, b3, wab, bab, bm):
  b, nd = x_num.shape
  grid = (b // bm,)

  def body(xn_ref, emb_ref, w1n_ref, w1c_ref, b1_ref, w2_ref, b2_ref,
           w3_ref, b3_ref, wab_ref, bab_ref, out_ref):
    h = jnp.dot(xn_ref[...], w1n_ref[...], preferred_element_type=jnp.float32)
    for g in range(4):
      h = h + jnp.dot(emb_ref[g], w1c_ref[g],
                      preferred_element_type=jnp.float32)
    h = jnp.maximum(h + b1_ref[...], 0.0)
    h = jnp.maximum(
        jnp.dot(h, w2_ref[...], preferred_element_type=jnp.float32)
        + b2_ref[...], 0.0)
    h = jnp.maximum(
        jnp.dot(h, w3_ref[...], preferred_element_type=jnp.float32)
        + b3_ref[...], 0.0)
    out_ref[...] = (
        jnp.dot(h, wab_ref[...], preferred_element_type=jnp.float32)
        + bab_ref[...])

  full2 = lambda shape: pl.BlockSpec(shape, lambda i: (0, 0))
  full3 = lambda shape: pl.BlockSpec(shape, lambda i: (0, 0, 0))
  return pl.pallas_call(
      body,
      grid=grid,
      in_specs=[
          pl.BlockSpec((bm, nd), lambda i: (i, 0)),
          pl.BlockSpec((4, bm, 128), lambda i: (0, i, 0)),
          full2(w1n.shape),
          full3(w1c.shape),
          full2(b1.shape),
          full2(w2.shape),
          full2(b2.shape),
          full2(w3.shape),
          full2(b3.shape),
          full2(wab.shape),
          full2(bab.shape),
      ],
      out_specs=pl.BlockSpec((bm, 2), lambda i: (i, 0)),
      out_shape=jax.ShapeDtypeStruct((b, 2), jnp.float32),
  )(x_num, emb3, w1n, w1c, b1, w2, b2, w3, b3, wab, bab)


_PLANE_FIELDS = (list(range(0, 8)),
                 [8, 9, 10, 11, 12, 0, 1, 2],
                 list(range(13, 21)),
                 [21, 22, 23, 24, 25, 13, 14, 15])
_REAL_SLOTS = (8, 5, 8, 5)


def kernel(x_num, x_cat, E, W1, b1, W2, b2, W3, b3, WA, bA, WB, bB):
  f, v, d = E.shape
  b = x_cat.shape[0]
  nd = x_num.shape[1]

  e_t = jnp.transpose(E, (0, 2, 1))
  e_tail = jnp.pad(e_t[:, :, _NFULL * _FULLW:],
                   ((0, 0), (0, 0), (0, _TAILW - (v - _NFULL * _FULLW))))

  idx_all = x_cat + (jnp.arange(f, dtype=jnp.int32) * _VP)[None, :]
  cols = jnp.asarray(sum(_PLANE_FIELDS, []), dtype=jnp.int32)
  idx3 = jnp.take(idx_all, cols, axis=1).reshape(b, 4, 8)
  idx3 = idx3.transpose(1, 0, 2).reshape(-1)

  emb, _ = _sc_fused(e_t, e_tail, idx3)
  emb3 = emb.reshape(4, b, 8 * d)

  w1e = W1[nd:]
  blocks = []
  r0 = 0
  for p in range(4):
    nreal = _REAL_SLOTS[p] * d
    blk = w1e[r0:r0 + nreal]
    r0 += nreal
    if nreal < 128:
      blk = jnp.pad(blk, ((0, 128 - nreal), (0, 0)))
    blocks.append(blk)
  w1c = jnp.stack(blocks)

  wab = jnp.concatenate([WA, WB], axis=1)
  bab = jnp.concatenate([bA, bB])[None, :]
  out = _tc_mlp(x_num, emb3, W1[:nd], w1c, b1[None, :], W2, b2[None, :],
                W3, b3[None, :], wab, bab, bm=2048)
  return out[:, 0], out[:, 1]


# speedup vs baseline: 1.4784x; 1.2488x over previous
import functools

import jax
import jax.numpy as jnp
from jax import lax
from jax.experimental import pallas as pl
from jax.experimental.pallas import tpu as pltpu
from jax.experimental.pallas import tpu_sc as plsc

_NUM_CORES = 2
_NUM_SUBCORES = 16
_NW = _NUM_CORES * _NUM_SUBCORES
_L = 16
_V = 100000
_VP = 100096
_ROWS_F = _VP // 8
_FULLW = 1024
_NFULL = 96
_TAILW = 1792
_CHUNK = 256


def _sc_fused(e_t, e_tail, idx):
  f = e_t.shape[0]
  fh = f // 2
  n, = idx.shape
  per_w = n // _NW
  n_chunks = per_w // _CHUNK

  mesh = plsc.VectorSubcoreMesh(core_axis_name="c", subcore_axis_name="s")

  @functools.partial(
      pl.kernel,
      out_type=[
          jax.ShapeDtypeStruct((n * 16 // 128, 128), jnp.float32),
          jax.ShapeDtypeStruct((f * _ROWS_F, 128), jnp.float32),
      ],
      mesh=mesh,
      scratch_types=[
          pltpu.VMEM((_L, _TAILW), jnp.float32),
          pltpu.VMEM((_L, _FULLW), jnp.float32),
          pltpu.VMEM((_TAILW // 8, 128), jnp.float32),
          pltpu.VMEM((_CHUNK,), jnp.int32),
          pltpu.VMEM((_CHUNK,), jnp.int32),
          pltpu.VMEM((_CHUNK,), jnp.int32),
          pltpu.VMEM((_CHUNK, 128), jnp.float32),
          pltpu.VMEM((_CHUNK // 8, 128), jnp.float32),
          pltpu.SemaphoreType.DMA,
          pltpu.SemaphoreType.DMA,
      ],
      compiler_params=pltpu.CompilerParams(use_tc_tiling_on_sc=True,
                                           needs_layout_passes=False),
  )
  def fused_kernel(et_hbm, etail_hbm, idx_hbm, out_hbm, tab_hbm,
                   ina_v, inb_v, line_v, idx_v, q_v, r_v, buf_v, outc_v,
                   sem_a, sem_b):
    c = lax.axis_index("c")
    s = lax.axis_index("s")
    lanes = lax.iota(jnp.int32, _L)

    def stage(src, dst, width, vcol0, fg, sem):
      a = pltpu.async_copy(
          src.at[fg, pl.ds(0, 8), pl.ds(vcol0, width)],
          dst.at[pl.ds(0, 8), pl.ds(0, width)], sem)
      b = pltpu.async_copy(
          src.at[fg, pl.ds(8, 8), pl.ds(vcol0, width)],
          dst.at[pl.ds(8, 8), pl.ds(0, width)], sem)
      return a, b

    def repack(src_v, width, row0):
      def grp(go, carry):
        g0 = go * 8
        vs = [plsc.load_gather(src_v,
                               [lanes, jnp.full((_L,), g0 + i, jnp.int32)])
              for i in range(8)]
        for i in range(8):
          line_v[go, pl.ds(i * 16, _L)] = vs[i]
        return carry

      lax.fori_loop(0, width // 8, grp, 0, unroll=2)
      pltpu.sync_copy(
          line_v.at[pl.ds(0, width // 8)],
          tab_hbm.at[pl.ds(pl.multiple_of(row0, 8), width // 8)])

    def field_body(fl, carry):
      fg = c * fh + fl
      frow = fg * _ROWS_F
      ha = stage(et_hbm, ina_v, _FULLW, (s + 0 * _NUM_SUBCORES) * _FULLW,
                 fg, sem_a)
      for k in range(_NFULL // _NUM_SUBCORES):
        task = s + k * _NUM_SUBCORES
        cur, nxt = (ina_v, inb_v) if k % 2 == 0 else (inb_v, ina_v)
        hn = None
        if k + 1 < _NFULL // _NUM_SUBCORES:
          hn = stage(et_hbm, nxt, _FULLW,
                     (s + (k + 1) * _NUM_SUBCORES) * _FULLW, fg,
                     sem_b if k % 2 == 0 else sem_a)
        ha[0].wait()
        ha[1].wait()
        repack(cur, _FULLW, frow + task * (_FULLW // 8))
        if hn is not None:
          ha = hn
      return carry

    lax.fori_loop(0, fh, field_body, 0)

    @pl.when(s < fh)
    def _tail():
      fg = c * fh + s
      ta, tb = stage(etail_hbm, ina_v, _TAILW, 0, fg, sem_a)
      ta.wait()
      tb.wait()
      repack(ina_v, _TAILW, fg * _ROWS_F + _NFULL * (_FULLW // 8))

    plsc.subcore_barrier()

    wid = c * _NUM_SUBCORES + s
    base = pl.multiple_of(wid * per_w, per_w)

    def chunk_body(ch, carry):
      off = base + ch * _CHUNK
      pltpu.sync_copy(idx_hbm.at[pl.ds(off, _CHUNK)], idx_v)

      def split_body(i, carry2):
        ix = idx_v[pl.ds(i * _L, _L)]
        q_v[pl.ds(i * _L, _L)] = lax.shift_right_logical(ix, 3)
        r_v[pl.ds(i * _L, _L)] = lax.shift_left(jnp.bitwise_and(ix, 7), 4)
        return carry2

      lax.fori_loop(0, _CHUNK // _L, split_body, 0)
      pltpu.async_copy(tab_hbm.at[q_v], buf_v, sem_a).wait()

      def group_body(g, carry2):
        i0 = g * _L
        ivec = lanes + i0
        rvec = r_v[pl.ds(i0, _L)]
        orow = lax.shift_right_logical(ivec, 3)
        ocol = lax.shift_left(jnp.bitwise_and(ivec, 7), 4)
        vals = [plsc.load_gather(buf_v, [ivec, rvec + w]) for w in range(16)]
        for w in range(16):
          plsc.store_scatter(outc_v, [orow, ocol + w], vals[w])
        return carry2

      lax.fori_loop(0, _CHUNK // _L, group_body, 0)
      pltpu.sync_copy(
          outc_v,
          out_hbm.at[pl.ds(pl.multiple_of((off * 16) // 128, _CHUNK // 8),
                           _CHUNK // 8)])
      return carry

    lax.fori_loop(0, n_chunks, chunk_body, 0)

  return fused_kernel(e_t, e_tail, idx)


def _tc_mlp(x_num, emb3, w1n, w1c, b1, w2, b2, w3, b3, wab, bab, bm):
  b, nd = x_num.shape
  grid = (b // bm,)

  def body(xn_ref, emb_ref, w1n_ref, w1c_ref, b1_ref, w2_ref, b2_ref,
           w3_ref, b3_ref, wab_ref, bab_ref, out_ref):
    h = jnp.dot(xn_ref[...], w1n_ref[...], preferred_element_type=jnp.float32)
    for g in range(4):
      h = h + jnp.dot(emb_ref[g], w1c_ref[g],
                      preferred_element_type=jnp.float32)
    h = jnp.maximum(h + b1_ref[...], 0.0)
    h = jnp.maximum(
        jnp.dot(h, w2_ref[...], preferred_element_type=jnp.float32)
        + b2_ref[...], 0.0)
    h = jnp.maximum(
        jnp.dot(h, w3_ref[...], preferred_element_type=jnp.float32)
        + b3_ref[...], 0.0)
    out_ref[...] = (
        jnp.dot(h, wab_ref[...], preferred_element_type=jnp.float32)
        + bab_ref[...])

  full2 = lambda shape: pl.BlockSpec(shape, lambda i: (0, 0))
  full3 = lambda shape: pl.BlockSpec(shape, lambda i: (0, 0, 0))
  return pl.pallas_call(
      body,
      grid=grid,
      in_specs=[
          pl.BlockSpec((bm, nd), lambda i: (i, 0)),
          pl.BlockSpec((4, bm, 128), lambda i: (0, i, 0)),
          full2(w1n.shape),
          full3(w1c.shape),
          full2(b1.shape),
          full2(w2.shape),
          full2(b2.shape),
          full2(w3.shape),
          full2(b3.shape),
          full2(wab.shape),
          full2(bab.shape),
      ],
      out_specs=pl.BlockSpec((bm, 2), lambda i: (i, 0)),
      out_shape=jax.ShapeDtypeStruct((b, 2), jnp.float32),
  )(x_num, emb3, w1n, w1c, b1, w2, b2, w3, b3, wab, bab)


_PLANE_FIELDS = (list(range(0, 8)),
                 [8, 9, 10, 11, 12, 0, 1, 2],
                 list(range(13, 21)),
                 [21, 22, 23, 24, 25, 13, 14, 15])
_REAL_SLOTS = (8, 5, 8, 5)


def kernel(x_num, x_cat, E, W1, b1, W2, b2, W3, b3, WA, bA, WB, bB):
  f, v, d = E.shape
  b = x_cat.shape[0]
  nd = x_num.shape[1]

  e_t = jnp.transpose(E, (0, 2, 1))
  e_tail = jnp.pad(e_t[:, :, _NFULL * _FULLW:],
                   ((0, 0), (0, 0), (0, _TAILW - (v - _NFULL * _FULLW))))

  idx_all = x_cat + (jnp.arange(f, dtype=jnp.int32) * _VP)[None, :]
  cols = jnp.asarray(sum(_PLANE_FIELDS, []), dtype=jnp.int32)
  idx3 = jnp.take(idx_all, cols, axis=1).reshape(b, 4, 8)
  idx3 = idx3.transpose(1, 0, 2).reshape(-1)

  emb, _ = _sc_fused(e_t, e_tail, idx3)
  emb3 = emb.reshape(4, b, 8 * d)

  w1e = W1[nd:]
  blocks = []
  r0 = 0
  for p in range(4):
    nreal = _REAL_SLOTS[p] * d
    blk = w1e[r0:r0 + nreal]
    r0 += nreal
    if nreal < 128:
      blk = jnp.pad(blk, ((0, 128 - nreal), (0, 0)))
    blocks.append(blk)
  w1c = jnp.stack(blocks)

  wab = jnp.concatenate([WA, WB], axis=1)
  bab = jnp.concatenate([bA, bB])[None, :]
  out = _tc_mlp(x_num, emb3, W1[:nd], w1c, b1[None, :], W2, b2[None, :],
                W3, b3[None, :], wab, bab, bm=2048)
  return out[:, 0], out[:, 1]


# speedup vs baseline: 1.6585x; 1.1218x over previous
import functools

import jax
import jax.numpy as jnp
from jax import lax
from jax.experimental import pallas as pl
from jax.experimental.pallas import tpu as pltpu
from jax.experimental.pallas import tpu_sc as plsc

_NUM_CORES = 2
_NUM_SUBCORES = 16
_NW = _NUM_CORES * _NUM_SUBCORES
_L = 16
_V = 100000
_VP = 100096
_ROWS_F = _VP // 8
_FULLW = 1024
_NFULL = 96
_TAILW = 1792
_CHUNK = 256


def _sc_fused(e_t, e_tail, idx):
  f = e_t.shape[0]
  fh = f // 2
  n, = idx.shape
  per_w = n // _NW
  n_chunks = per_w // _CHUNK

  mesh = plsc.VectorSubcoreMesh(core_axis_name="c", subcore_axis_name="s")

  @functools.partial(
      pl.kernel,
      out_type=[
          jax.ShapeDtypeStruct((n * 16 // 128, 128), jnp.float32),
          jax.ShapeDtypeStruct((f * _ROWS_F, 128), jnp.float32),
      ],
      mesh=mesh,
      scratch_types=[
          pltpu.VMEM((_L, _FULLW), jnp.float32),
          pltpu.VMEM((_L, _FULLW), jnp.float32),
          pltpu.VMEM((_FULLW // 8, 128), jnp.float32),
          pltpu.VMEM((_CHUNK,), jnp.int32),
          pltpu.VMEM((_CHUNK,), jnp.int32),
          pltpu.VMEM((_CHUNK,), jnp.int32),
          pltpu.VMEM((_CHUNK,), jnp.int32),
          pltpu.VMEM((_CHUNK,), jnp.int32),
          pltpu.VMEM((_CHUNK,), jnp.int32),
          pltpu.VMEM((_CHUNK, 128), jnp.float32),
          pltpu.VMEM((_CHUNK, 128), jnp.float32),
          pltpu.VMEM((_CHUNK // 8, 128), jnp.float32),
          pltpu.SemaphoreType.DMA,
          pltpu.SemaphoreType.DMA,
      ],
      compiler_params=pltpu.CompilerParams(use_tc_tiling_on_sc=True,
                                           needs_layout_passes=False),
  )
  def fused_kernel(et_hbm, etail_hbm, idx_hbm, out_hbm, tab_hbm,
                   ina_v, inb_v, line_v, idx0_v, idx1_v, q0_v, q1_v,
                   r0_v, r1_v, buf0_v, buf1_v, outc_v, sem_a, sem_b):
    idx_vs = (idx0_v, idx1_v)
    q_vs = (q0_v, q1_v)
    r_vs = (r0_v, r1_v)
    buf_vs = (buf0_v, buf1_v)
    c = lax.axis_index("c")
    s = lax.axis_index("s")
    lanes = lax.iota(jnp.int32, _L)

    def stage(src, dst, width, vcol0, fg, sem):
      a = pltpu.async_copy(
          src.at[fg, pl.ds(0, 8), pl.ds(vcol0, width)],
          dst.at[pl.ds(0, 8), pl.ds(0, width)], sem)
      b = pltpu.async_copy(
          src.at[fg, pl.ds(8, 8), pl.ds(vcol0, width)],
          dst.at[pl.ds(8, 8), pl.ds(0, width)], sem)
      return a, b

    def repack(src_v, width, row0):
      def grp(go, carry):
        g0 = go * 8
        vs = [plsc.load_gather(src_v,
                               [lanes, jnp.full((_L,), g0 + i, jnp.int32)])
              for i in range(8)]
        for i in range(8):
          line_v[go, pl.ds(i * 16, _L)] = vs[i]
        return carry

      lax.fori_loop(0, width // 8, grp, 0, unroll=2)
      pltpu.sync_copy(
          line_v.at[pl.ds(0, width // 8)],
          tab_hbm.at[pl.ds(pl.multiple_of(row0, 8), width // 8)])

    def field_body(fl, carry):
      fg = c * fh + fl
      frow = fg * _ROWS_F
      ha = stage(et_hbm, ina_v, _FULLW, (s + 0 * _NUM_SUBCORES) * _FULLW,
                 fg, sem_a)
      for k in range(_NFULL // _NUM_SUBCORES):
        task = s + k * _NUM_SUBCORES
        cur, nxt = (ina_v, inb_v) if k % 2 == 0 else (inb_v, ina_v)
        hn = None
        if k + 1 < _NFULL // _NUM_SUBCORES:
          hn = stage(et_hbm, nxt, _FULLW,
                     (s + (k + 1) * _NUM_SUBCORES) * _FULLW, fg,
                     sem_b if k % 2 == 0 else sem_a)
        ha[0].wait()
        ha[1].wait()
        repack(cur, _FULLW, frow + task * (_FULLW // 8))
        if hn is not None:
          ha = hn
      return carry

    lax.fori_loop(0, fh, field_body, 0)

    @pl.when(s < fh)
    def _tail():
      fg = c * fh + s
      for part in range(2):
        ta, tb = stage(etail_hbm, ina_v, _TAILW // 2, part * (_TAILW // 2),
                       fg, sem_a)
        ta.wait()
        tb.wait()
        repack(ina_v, _TAILW // 2,
               fg * _ROWS_F + _NFULL * (_FULLW // 8) + part * (_TAILW // 16))

    plsc.subcore_barrier()

    wid = c * _NUM_SUBCORES + s
    base = pl.multiple_of(wid * per_w, per_w)
    sems = (sem_a, sem_b)

    def stage_chunk(ci, k):
      off = base + ci * _CHUNK
      pltpu.sync_copy(idx_hbm.at[pl.ds(off, _CHUNK)], idx_vs[k])

      def split_body(i, carry2):
        ix = idx_vs[k][pl.ds(i * _L, _L)]
        q_vs[k][pl.ds(i * _L, _L)] = lax.shift_right_logical(ix, 3)
        r_vs[k][pl.ds(i * _L, _L)] = lax.shift_left(
            jnp.bitwise_and(ix, 7), 4)
        return carry2

      lax.fori_loop(0, _CHUNK // _L, split_body, 0)
      pltpu.async_copy(tab_hbm.at[q_vs[k]], buf_vs[k], sems[k])

    def wait_chunk(k):
      pltpu.make_async_copy(tab_hbm.at[q_vs[k]], buf_vs[k], sems[k]).wait()

    def compact_chunk(ci, k):
      def group_body(g, carry2):
        i0 = g * _L
        ivec = lanes + i0
        rvec = r_vs[k][pl.ds(i0, _L)]
        orow = lax.shift_right_logical(ivec, 3)
        ocol = lax.shift_left(jnp.bitwise_and(ivec, 7), 4)
        vals = [plsc.load_gather(buf_vs[k], [ivec, rvec + w])
                for w in range(16)]
        for w in range(16):
          plsc.store_scatter(outc_v, [orow, ocol + w], vals[w])
        return carry2

      lax.fori_loop(0, _CHUNK // _L, group_body, 0)
      orow0 = (base + ci * _CHUNK) // 8
      pltpu.sync_copy(
          outc_v,
          out_hbm.at[pl.ds(pl.multiple_of(orow0, _CHUNK // 8), _CHUNK // 8)])

    stage_chunk(0, 0)

    def pair_body(i, carry):
      c0 = i * 2
      stage_chunk(c0 + 1, 1)
      wait_chunk(0)
      compact_chunk(c0, 0)
      stage_chunk(lax.rem(c0 + 2, n_chunks), 0)
      wait_chunk(1)
      compact_chunk(c0 + 1, 1)
      return carry

    lax.fori_loop(0, n_chunks // 2, pair_body, 0)
    wait_chunk(0)

  return fused_kernel(e_t, e_tail, idx)


def _tc_mlp(x_num, emb3, w1n, w1c, b1, w2, b2, w3, b3, wab, bab, bm):
  b, nd = x_num.shape
  grid = (b // bm,)

  def body(xn_ref, emb_ref, w1n_ref, w1c_ref, b1_ref, w2_ref, b2_ref,
           w3_ref, b3_ref, wab_ref, bab_ref, out_ref):
    h = jnp.dot(xn_ref[...], w1n_ref[...], preferred_element_type=jnp.float32)
    for g in range(4):
      h = h + jnp.dot(emb_ref[g], w1c_ref[g],
                      preferred_element_type=jnp.float32)
    h = jnp.maximum(h + b1_ref[...], 0.0)
    h = jnp.maximum(
        jnp.dot(h, w2_ref[...], preferred_element_type=jnp.float32)
        + b2_ref[...], 0.0)
    h = jnp.maximum(
        jnp.dot(h, w3_ref[...], preferred_element_type=jnp.float32)
        + b3_ref[...], 0.0)
    out_ref[...] = (
        jnp.dot(h, wab_ref[...], preferred_element_type=jnp.float32)
        + bab_ref[...])

  full2 = lambda shape: pl.BlockSpec(shape, lambda i: (0, 0))
  full3 = lambda shape: pl.BlockSpec(shape, lambda i: (0, 0, 0))
  return pl.pallas_call(
      body,
      grid=grid,
      in_specs=[
          pl.BlockSpec((bm, nd), lambda i: (i, 0)),
          pl.BlockSpec((4, bm, 128), lambda i: (0, i, 0)),
          full2(w1n.shape),
          full3(w1c.shape),
          full2(b1.shape),
          full2(w2.shape),
          full2(b2.shape),
          full2(w3.shape),
          full2(b3.shape),
          full2(wab.shape),
          full2(bab.shape),
      ],
      out_specs=pl.BlockSpec((bm, 2), lambda i: (i, 0)),
      out_shape=jax.ShapeDtypeStruct((b, 2), jnp.float32),
  )(x_num, emb3, w1n, w1c, b1, w2, b2, w3, b3, wab, bab)


_PLANE_FIELDS = (list(range(0, 8)),
                 [8, 9, 10, 11, 12, 0, 1, 2],
                 list(range(13, 21)),
                 [21, 22, 23, 24, 25, 13, 14, 15])
_REAL_SLOTS = (8, 5, 8, 5)


def kernel(x_num, x_cat, E, W1, b1, W2, b2, W3, b3, WA, bA, WB, bB):
  f, v, d = E.shape
  b = x_cat.shape[0]
  nd = x_num.shape[1]

  e_t = jnp.transpose(E, (0, 2, 1))
  e_tail = jnp.pad(e_t[:, :, _NFULL * _FULLW:],
                   ((0, 0), (0, 0), (0, _TAILW - (v - _NFULL * _FULLW))))

  idx_all = x_cat + (jnp.arange(f, dtype=jnp.int32) * _VP)[None, :]
  cols = jnp.asarray(sum(_PLANE_FIELDS, []), dtype=jnp.int32)
  idx3 = jnp.take(idx_all, cols, axis=1).reshape(b, 4, 8)
  idx3 = idx3.transpose(1, 0, 2).reshape(-1)

  emb, _ = _sc_fused(e_t, e_tail, idx3)
  emb3 = emb.reshape(4, b, 8 * d)

  w1e = W1[nd:]
  blocks = []
  r0 = 0
  for p in range(4):
    nreal = _REAL_SLOTS[p] * d
    blk = w1e[r0:r0 + nreal]
    r0 += nreal
    if nreal < 128:
      blk = jnp.pad(blk, ((0, 128 - nreal), (0, 0)))
    blocks.append(blk)
  w1c = jnp.stack(blocks)

  wab = jnp.concatenate([WA, WB], axis=1)
  bab = jnp.concatenate([bA, bB])[None, :]
  out = _tc_mlp(x_num, emb3, W1[:nd], w1c, b1[None, :], W2, b2[None, :],
                W3, b3[None, :], wab, bab, bm=2048)
  return out[:, 0], out[:, 1]


# speedup vs baseline: 1.6906x; 1.0194x over previous
import functools

import jax
import jax.numpy as jnp
from jax import lax
from jax.experimental import pallas as pl
from jax.experimental.pallas import tpu as pltpu
from jax.experimental.pallas import tpu_sc as plsc

_NUM_CORES = 2
_NUM_SUBCORES = 16
_NW = _NUM_CORES * _NUM_SUBCORES
_L = 16
_V = 100000
_VP = 100096
_ROWS_F = _VP // 8
_FULLW = 1024
_NFULL = 96
_TAILW = 1792
_CHUNK = 256


def _sc_fused(e_t, e_tail, idx):
  f = e_t.shape[0]
  fh = f // 2
  n, = idx.shape
  per_w = n // _NW
  n_chunks = per_w // _CHUNK

  mesh = plsc.VectorSubcoreMesh(core_axis_name="c", subcore_axis_name="s")

  @functools.partial(
      pl.kernel,
      out_type=[
          jax.ShapeDtypeStruct((n * 16 // 128, 128), jnp.float32),
          jax.ShapeDtypeStruct((f * _ROWS_F, 128), jnp.float32),
      ],
      mesh=mesh,
      scratch_types=[
          pltpu.VMEM((_L, _FULLW), jnp.float32),
          pltpu.VMEM((_L, _FULLW), jnp.float32),
          pltpu.VMEM((_FULLW // 8, 128), jnp.float32),
          pltpu.VMEM((_CHUNK,), jnp.int32),
          pltpu.VMEM((_CHUNK,), jnp.int32),
          pltpu.VMEM((_CHUNK,), jnp.int32),
          pltpu.VMEM((_CHUNK,), jnp.int32),
          pltpu.VMEM((_CHUNK,), jnp.int32),
          pltpu.VMEM((_CHUNK,), jnp.int32),
          pltpu.VMEM((_CHUNK, 128), jnp.float32),
          pltpu.VMEM((_CHUNK, 128), jnp.float32),
          pltpu.VMEM((_CHUNK // 8, 128), jnp.float32),
          pltpu.SemaphoreType.DMA,
          pltpu.SemaphoreType.DMA,
          pltpu.SemaphoreType.DMA,
          pltpu.SemaphoreType.DMA,
      ],
      compiler_params=pltpu.CompilerParams(use_tc_tiling_on_sc=True,
                                           needs_layout_passes=False),
  )
  def fused_kernel(et_hbm, etail_hbm, idx_hbm, out_hbm, tab_hbm,
                   ina_v, inb_v, line_v, idx0_v, idx1_v, q0_v, q1_v,
                   r0_v, r1_v, buf0_v, buf1_v, outc_v, sem_a, sem_b,
                   sem_c, sem_d):
    idx_vs = (idx0_v, idx1_v)
    q_vs = (q0_v, q1_v)
    r_vs = (r0_v, r1_v)
    buf_vs = (buf0_v, buf1_v)
    c = lax.axis_index("c")
    s = lax.axis_index("s")
    lanes = lax.iota(jnp.int32, _L)

    def stage(src, dst, width, vcol0, fg, sem):
      a = pltpu.async_copy(
          src.at[fg, pl.ds(0, 8), pl.ds(vcol0, width)],
          dst.at[pl.ds(0, 8), pl.ds(0, width)], sem)
      b = pltpu.async_copy(
          src.at[fg, pl.ds(8, 8), pl.ds(vcol0, width)],
          dst.at[pl.ds(8, 8), pl.ds(0, width)], sem)
      return a, b

    def repack(src_v, width, row0):
      def grp(go, carry):
        g0 = go * 8
        vs = [plsc.load_gather(src_v,
                               [lanes, jnp.full((_L,), g0 + i, jnp.int32)])
              for i in range(8)]
        for i in range(8):
          line_v[go, pl.ds(i * 16, _L)] = vs[i]
        return carry

      lax.fori_loop(0, width // 8, grp, 0, unroll=2)
      pltpu.sync_copy(
          line_v.at[pl.ds(0, width // 8)],
          tab_hbm.at[pl.ds(pl.multiple_of(row0, 8), width // 8)])

    def field_body(fl, carry):
      fg = c * fh + fl
      frow = fg * _ROWS_F
      ha = stage(et_hbm, ina_v, _FULLW, (s + 0 * _NUM_SUBCORES) * _FULLW,
                 fg, sem_a)
      for k in range(_NFULL // _NUM_SUBCORES):
        task = s + k * _NUM_SUBCORES
        cur, nxt = (ina_v, inb_v) if k % 2 == 0 else (inb_v, ina_v)
        hn = None
        if k + 1 < _NFULL // _NUM_SUBCORES:
          hn = stage(et_hbm, nxt, _FULLW,
                     (s + (k + 1) * _NUM_SUBCORES) * _FULLW, fg,
                     sem_b if k % 2 == 0 else sem_a)
        ha[0].wait()
        ha[1].wait()
        repack(cur, _FULLW, frow + task * (_FULLW // 8))
        if hn is not None:
          ha = hn
      return carry

    lax.fori_loop(0, fh, field_body, 0)

    @pl.when(s < fh)
    def _tail():
      fg = c * fh + s
      for part in range(2):
        ta, tb = stage(etail_hbm, ina_v, _TAILW // 2, part * (_TAILW // 2),
                       fg, sem_a)
        ta.wait()
        tb.wait()
        repack(ina_v, _TAILW // 2,
               fg * _ROWS_F + _NFULL * (_FULLW // 8) + part * (_TAILW // 16))

    plsc.subcore_barrier()

    wid = c * _NUM_SUBCORES + s
    base = pl.multiple_of(wid * per_w, per_w)
    sems = (sem_a, sem_b)

    isems = (sem_c, sem_d)

    def fire_idx(ci, k):
      off = base + ci * _CHUNK
      pltpu.async_copy(idx_hbm.at[pl.ds(off, _CHUNK)], idx_vs[k], isems[k])

    def launch(ci, k):
      off = base + ci * _CHUNK
      pltpu.make_async_copy(idx_hbm.at[pl.ds(off, _CHUNK)], idx_vs[k],
                            isems[k]).wait()

      def split_body(i, carry2):
        ix = idx_vs[k][pl.ds(i * _L, _L)]
        q_vs[k][pl.ds(i * _L, _L)] = lax.shift_right_logical(ix, 3)
        r_vs[k][pl.ds(i * _L, _L)] = lax.shift_left(
            jnp.bitwise_and(ix, 7), 4)
        return carry2

      lax.fori_loop(0, _CHUNK // _L, split_body, 0)
      pltpu.async_copy(tab_hbm.at[q_vs[k]], buf_vs[k], sems[k])

    def wait_chunk(k):
      pltpu.make_async_copy(tab_hbm.at[q_vs[k]], buf_vs[k], sems[k]).wait()

    def compact_chunk(ci, k):
      def group_body(g, carry2):
        i0 = g * _L
        ivec = lanes + i0
        rvec = r_vs[k][pl.ds(i0, _L)]
        orow = lax.shift_right_logical(ivec, 3)
        ocol = lax.shift_left(jnp.bitwise_and(ivec, 7), 4)
        vals = [plsc.load_gather(buf_vs[k], [ivec, rvec + w])
                for w in range(16)]
        for w in range(16):
          plsc.store_scatter(outc_v, [orow, ocol + w], vals[w])
        return carry2

      lax.fori_loop(0, _CHUNK // _L, group_body, 0)
      orow0 = (base + ci * _CHUNK) // 8
      pltpu.sync_copy(
          outc_v,
          out_hbm.at[pl.ds(pl.multiple_of(orow0, _CHUNK // 8), _CHUNK // 8)])

    fire_idx(0, 0)
    fire_idx(1, 1)
    launch(0, 0)

    def pair_body(i, carry):
      c0 = i * 2
      launch(c0 + 1, 1)
      fire_idx(lax.rem(c0 + 2, n_chunks), 0)
      wait_chunk(0)
      compact_chunk(c0, 0)
      launch(lax.rem(c0 + 2, n_chunks), 0)
      fire_idx(lax.rem(c0 + 3, n_chunks), 1)
      wait_chunk(1)
      compact_chunk(c0 + 1, 1)
      return carry

    lax.fori_loop(0, n_chunks // 2, pair_body, 0)
    wait_chunk(0)
    pltpu.make_async_copy(idx_hbm.at[pl.ds(base + _CHUNK, _CHUNK)],
                          idx_vs[1], isems[1]).wait()

  return fused_kernel(e_t, e_tail, idx)


def _tc_mlp(x_num, emb3, w1n, w1c, b1, w2, b2, w3, b3, wab, bab, bm):
  b, nd = x_num.shape
  grid = (b // bm,)

  def body(xn_ref, emb_ref, w1n_ref, w1c_ref, b1_ref, w2_ref, b2_ref,
           w3_ref, b3_ref, wab_ref, bab_ref, out_ref):
    h = jnp.dot(xn_ref[...], w1n_ref[...], preferred_element_type=jnp.float32)
    for g in range(4):
      h = h + jnp.dot(emb_ref[g], w1c_ref[g],
                      preferred_element_type=jnp.float32)
    h = jnp.maximum(h + b1_ref[...], 0.0)
    h = jnp.maximum(
        jnp.dot(h, w2_ref[...], preferred_element_type=jnp.float32)
        + b2_ref[...], 0.0)
    h = jnp.maximum(
        jnp.dot(h, w3_ref[...], preferred_element_type=jnp.float32)
        + b3_ref[...], 0.0)
    out_ref[...] = (
        jnp.dot(h, wab_ref[...], preferred_element_type=jnp.float32)
        + bab_ref[...])

  full2 = lambda shape: pl.BlockSpec(shape, lambda i: (0, 0))
  full3 = lambda shape: pl.BlockSpec(shape, lambda i: (0, 0, 0))
  return pl.pallas_call(
      body,
      grid=grid,
      in_specs=[
          pl.BlockSpec((bm, nd), lambda i: (i, 0)),
          pl.BlockSpec((4, bm, 128), lambda i: (0, i, 0)),
          full2(w1n.shape),
          full3(w1c.shape),
          full2(b1.shape),
          full2(w2.shape),
          full2(b2.shape),
          full2(w3.shape),
          full2(b3.shape),
          full2(wab.shape),
          full2(bab.shape),
      ],
      out_specs=pl.BlockSpec((bm, 2), lambda i: (i, 0)),
      out_shape=jax.ShapeDtypeStruct((b, 2), jnp.float32),
  )(x_num, emb3, w1n, w1c, b1, w2, b2, w3, b3, wab, bab)


_PLANE_FIELDS = (list(range(0, 8)),
                 [8, 9, 10, 11, 12, 0, 1, 2],
                 list(range(13, 21)),
                 [21, 22, 23, 24, 25, 13, 14, 15])
_REAL_SLOTS = (8, 5, 8, 5)


def kernel(x_num, x_cat, E, W1, b1, W2, b2, W3, b3, WA, bA, WB, bB):
  f, v, d = E.shape
  b = x_cat.shape[0]
  nd = x_num.shape[1]

  e_t = jnp.transpose(E, (0, 2, 1))
  e_tail = jnp.pad(e_t[:, :, _NFULL * _FULLW:],
                   ((0, 0), (0, 0), (0, _TAILW - (v - _NFULL * _FULLW))))

  idx_all = x_cat + (jnp.arange(f, dtype=jnp.int32) * _VP)[None, :]
  cols = jnp.asarray(sum(_PLANE_FIELDS, []), dtype=jnp.int32)
  idx3 = jnp.take(idx_all, cols, axis=1).reshape(b, 4, 8)
  idx3 = idx3.transpose(1, 0, 2).reshape(-1)

  emb, _ = _sc_fused(e_t, e_tail, idx3)
  emb3 = emb.reshape(4, b, 8 * d)

  w1e = W1[nd:]
  blocks = []
  r0 = 0
  for p in range(4):
    nreal = _REAL_SLOTS[p] * d
    blk = w1e[r0:r0 + nreal]
    r0 += nreal
    if nreal < 128:
      blk = jnp.pad(blk, ((0, 128 - nreal), (0, 0)))
    blocks.append(blk)
  w1c = jnp.stack(blocks)

  wab = jnp.concatenate([WA, WB], axis=1)
  bab = jnp.concatenate([bA, bB])[None, :]
  out = _tc_mlp(x_num, emb3, W1[:nd], w1c, b1[None, :], W2, b2[None, :],
                W3, b3[None, :], wab, bab, bm=2048)
  return out[:, 0], out[:, 1]


# speedup vs baseline: 1.8040x; 1.0671x over previous
import functools

import jax
import jax.numpy as jnp
from jax import lax
from jax.experimental import pallas as pl
from jax.experimental.pallas import tpu as pltpu
from jax.experimental.pallas import tpu_sc as plsc

_NUM_CORES = 2
_NUM_SUBCORES = 16
_NW = _NUM_CORES * _NUM_SUBCORES
_L = 16
_V = 100000
_VP = 100096
_ROWS_F = _VP // 8
_FULLW = 1024
_NFULL = 96
_TAILW = 1792
_CHUNK = 256


def _sc_fused(e_t, e_tail, idx):
  f = e_t.shape[0]
  fh = f // 2
  n, = idx.shape
  per_w = n // _NW
  n_chunks = per_w // _CHUNK

  mesh = plsc.VectorSubcoreMesh(core_axis_name="c", subcore_axis_name="s")

  @functools.partial(
      pl.kernel,
      out_type=[
          jax.ShapeDtypeStruct((n * 16 // 128, 128), jnp.float32),
          jax.ShapeDtypeStruct((f * _ROWS_F, 128), jnp.float32),
      ],
      mesh=mesh,
      scratch_types=[
          pltpu.SemaphoreType.DMA,
          pltpu.SemaphoreType.DMA,
          pltpu.SemaphoreType.DMA,
          pltpu.SemaphoreType.DMA,
      ],
      compiler_params=pltpu.CompilerParams(use_tc_tiling_on_sc=True,
                                           needs_layout_passes=False),
  )
  def fused_kernel(et_hbm, etail_hbm, idx_hbm, out_hbm, tab_hbm,
                   sem_a, sem_b, sem_c, sem_d):
    c = lax.axis_index("c")
    s = lax.axis_index("s")
    lanes = lax.iota(jnp.int32, _L)

    def stage(src, dst, width, vcol0, fg, sem):
      a = pltpu.async_copy(
          src.at[fg, pl.ds(0, 8), pl.ds(vcol0, width)],
          dst.at[pl.ds(0, 8), pl.ds(0, width)], sem)
      b = pltpu.async_copy(
          src.at[fg, pl.ds(8, 8), pl.ds(vcol0, width)],
          dst.at[pl.ds(8, 8), pl.ds(0, width)], sem)
      return a, b

    def phase1(ina_v, inb_v, linea_v, lineb_v):
      def repack(src_v, line_v, width, row0, prev):
        if prev is not None:
          prev.wait()

        def grp(go, carry):
          g0 = go * 8
          vs = [plsc.load_gather(src_v,
                                 [lanes, jnp.full((_L,), g0 + i, jnp.int32)])
                for i in range(8)]
          for i in range(8):
            line_v[go, pl.ds(i * 16, _L)] = vs[i]
          return carry

        lax.fori_loop(0, width // 8, grp, 0, unroll=2)
        return pltpu.async_copy(
            line_v.at[pl.ds(0, width // 8)],
            tab_hbm.at[pl.ds(pl.multiple_of(row0, 8), width // 8)], sem_c)

      def field_body(fl, carry):
        fg = c * fh + fl
        frow = fg * _ROWS_F
        ha = stage(et_hbm, ina_v, _FULLW, s * _FULLW, fg, sem_a)
        pend = [None, None]
        for k in range(_NFULL // _NUM_SUBCORES):
          task = s + k * _NUM_SUBCORES
          cur = ina_v if k % 2 == 0 else inb_v
          line = linea_v if k % 2 == 0 else lineb_v
          hn = None
          if k + 1 < _NFULL // _NUM_SUBCORES:
            hn = stage(et_hbm, inb_v if k % 2 == 0 else ina_v, _FULLW,
                       (s + (k + 1) * _NUM_SUBCORES) * _FULLW, fg,
                       sem_b if k % 2 == 0 else sem_a)
          ha[0].wait()
          ha[1].wait()
          pend[k % 2] = repack(cur, line, _FULLW,
                               frow + task * (_FULLW // 8), pend[k % 2])
          if hn is not None:
            ha = hn
        pend[0].wait()
        pend[1].wait()
        return carry

      lax.fori_loop(0, fh, field_body, 0)

      @pl.when(s < fh)
      def _tail():
        fg = c * fh + s
        for part in range(2):
          ta, tb = stage(etail_hbm, ina_v, _TAILW // 2,
                         part * (_TAILW // 2), fg, sem_a)
          ta.wait()
          tb.wait()
          h = repack(ina_v, linea_v, _TAILW // 2,
                     fg * _ROWS_F + _NFULL * (_FULLW // 8)
                     + part * (_TAILW // 16), None)
          h.wait()

    pl.run_scoped(phase1,
                  pltpu.VMEM((_L, _FULLW), jnp.float32),
                  pltpu.VMEM((_L, _FULLW), jnp.float32),
                  pltpu.VMEM((_FULLW // 8, 128), jnp.float32),
                  pltpu.VMEM((_FULLW // 8, 128), jnp.float32))

    plsc.subcore_barrier()

    wid = c * _NUM_SUBCORES + s
    base = pl.multiple_of(wid * per_w, per_w)

    def phase2(idx0_v, idx1_v, q0_v, q1_v, r0_v, r1_v, buf0_v, buf1_v,
               outc_v):
      idx_vs = (idx0_v, idx1_v)
      q_vs = (q0_v, q1_v)
      r_vs = (r0_v, r1_v)
      buf_vs = (buf0_v, buf1_v)
      sems = (sem_a, sem_b)
      isems = (sem_c, sem_d)

      def fire_idx(ci, k):
        off = base + ci * _CHUNK
        pltpu.async_copy(idx_hbm.at[pl.ds(off, _CHUNK)], idx_vs[k],
                         isems[k])

      def launch(ci, k):
        off = base + ci * _CHUNK
        pltpu.make_async_copy(idx_hbm.at[pl.ds(off, _CHUNK)], idx_vs[k],
                              isems[k]).wait()

        def split_body(i, carry2):
          ix = idx_vs[k][pl.ds(i * _L, _L)]
          q_vs[k][pl.ds(i * _L, _L)] = lax.shift_right_logical(ix, 3)
          r_vs[k][pl.ds(i * _L, _L)] = lax.shift_left(
              jnp.bitwise_and(ix, 7), 4)
          return carry2

        lax.fori_loop(0, _CHUNK // _L, split_body, 0)
        pltpu.async_copy(tab_hbm.at[q_vs[k]], buf_vs[k], sems[k])

      def wait_chunk(k):
        pltpu.make_async_copy(tab_hbm.at[q_vs[k]], buf_vs[k],
                              sems[k]).wait()

      def compact_chunk(ci, k):
        def group_body(g, carry2):
          i0 = g * _L
          ivec = lanes + i0
          rvec = r_vs[k][pl.ds(i0, _L)]
          orow = lax.shift_right_logical(ivec, 3)
          ocol = lax.shift_left(jnp.bitwise_and(ivec, 7), 4)
          vals = [plsc.load_gather(buf_vs[k], [ivec, rvec + w])
                  for w in range(16)]
          for w in range(16):
            plsc.store_scatter(outc_v, [orow, ocol + w], vals[w])
          return carry2

        lax.fori_loop(0, _CHUNK // _L, group_body, 0)
        orow0 = (base + ci * _CHUNK) // 8
        pltpu.sync_copy(
            outc_v,
            out_hbm.at[pl.ds(pl.multiple_of(orow0, _CHUNK // 8),
                             _CHUNK // 8)])

      fire_idx(0, 0)
      fire_idx(1, 1)
      launch(0, 0)

      def pair_body(i, carry):
        c0 = i * 2
        launch(c0 + 1, 1)
        fire_idx(lax.rem(c0 + 2, n_chunks), 0)
        wait_chunk(0)
        compact_chunk(c0, 0)
        launch(lax.rem(c0 + 2, n_chunks), 0)
        fire_idx(lax.rem(c0 + 3, n_chunks), 1)
        wait_chunk(1)
        compact_chunk(c0 + 1, 1)
        return carry

      lax.fori_loop(0, n_chunks // 2, pair_body, 0)
      wait_chunk(0)
      pltpu.make_async_copy(idx_hbm.at[pl.ds(base + _CHUNK, _CHUNK)],
                            idx_vs[1], isems[1]).wait()

    pl.run_scoped(phase2,
                  pltpu.VMEM((_CHUNK,), jnp.int32),
                  pltpu.VMEM((_CHUNK,), jnp.int32),
                  pltpu.VMEM((_CHUNK,), jnp.int32),
                  pltpu.VMEM((_CHUNK,), jnp.int32),
                  pltpu.VMEM((_CHUNK,), jnp.int32),
                  pltpu.VMEM((_CHUNK,), jnp.int32),
                  pltpu.VMEM((_CHUNK, 128), jnp.float32),
                  pltpu.VMEM((_CHUNK, 128), jnp.float32),
                  pltpu.VMEM((_CHUNK // 8, 128), jnp.float32))

  return fused_kernel(e_t, e_tail, idx)


def _tc_mlp(x_num, emb3, w1n, w1c, b1, w2, b2, w3, b3, wab, bab, bm):
  b, nd = x_num.shape
  grid = (b // bm,)

  def body(xn_ref, emb_ref, w1n_ref, w1c_ref, b1_ref, w2_ref, b2_ref,
           w3_ref, b3_ref, wab_ref, bab_ref, out_ref):
    h = jnp.dot(xn_ref[...], w1n_ref[...], preferred_element_type=jnp.float32)
    for g in range(4):
      h = h + jnp.dot(emb_ref[g], w1c_ref[g],
                      preferred_element_type=jnp.float32)
    h = jnp.maximum(h + b1_ref[...], 0.0)
    h = jnp.maximum(
        jnp.dot(h, w2_ref[...], preferred_element_type=jnp.float32)
        + b2_ref[...], 0.0)
    h = jnp.maximum(
        jnp.dot(h, w3_ref[...], preferred_element_type=jnp.float32)
        + b3_ref[...], 0.0)
    out_ref[...] = (
        jnp.dot(h, wab_ref[...], preferred_element_type=jnp.float32)
        + bab_ref[...])

  full2 = lambda shape: pl.BlockSpec(shape, lambda i: (0, 0))
  full3 = lambda shape: pl.BlockSpec(shape, lambda i: (0, 0, 0))
  return pl.pallas_call(
      body,
      grid=grid,
      in_specs=[
          pl.BlockSpec((bm, nd), lambda i: (i, 0)),
          pl.BlockSpec((4, bm, 128), lambda i: (0, i, 0)),
          full2(w1n.shape),
          full3(w1c.shape),
          full2(b1.shape),
          full2(w2.shape),
          full2(b2.shape),
          full2(w3.shape),
          full2(b3.shape),
          full2(wab.shape),
          full2(bab.shape),
      ],
      out_specs=pl.BlockSpec((bm, 2), lambda i: (i, 0)),
      out_shape=jax.ShapeDtypeStruct((b, 2), jnp.float32),
  )(x_num, emb3, w1n, w1c, b1, w2, b2, w3, b3, wab, bab)


_PLANE_FIELDS = (list(range(0, 8)),
                 [8, 9, 10, 11, 12, 0, 1, 2],
                 list(range(13, 21)),
                 [21, 22, 23, 24, 25, 13, 14, 15])
_REAL_SLOTS = (8, 5, 8, 5)


def kernel(x_num, x_cat, E, W1, b1, W2, b2, W3, b3, WA, bA, WB, bB):
  f, v, d = E.shape
  b = x_cat.shape[0]
  nd = x_num.shape[1]

  e_t = jnp.transpose(E, (0, 2, 1))
  e_tail = jnp.pad(e_t[:, :, _NFULL * _FULLW:],
                   ((0, 0), (0, 0), (0, _TAILW - (v - _NFULL * _FULLW))))

  idx_all = x_cat + (jnp.arange(f, dtype=jnp.int32) * _VP)[None, :]
  cols = jnp.asarray(sum(_PLANE_FIELDS, []), dtype=jnp.int32)
  idx3 = jnp.take(idx_all, cols, axis=1).reshape(b, 4, 8)
  idx3 = idx3.transpose(1, 0, 2).reshape(-1)

  emb, _ = _sc_fused(e_t, e_tail, idx3)
  emb3 = emb.reshape(4, b, 8 * d)

  w1e = W1[nd:]
  blocks = []
  r0 = 0
  for p in range(4):
    nreal = _REAL_SLOTS[p] * d
    blk = w1e[r0:r0 + nreal]
    r0 += nreal
    if nreal < 128:
      blk = jnp.pad(blk, ((0, 128 - nreal), (0, 0)))
    blocks.append(blk)
  w1c = jnp.stack(blocks)

  wab = jnp.concatenate([WA, WB], axis=1)
  bab = jnp.concatenate([bA, bB])[None, :]
  out = _tc_mlp(x_num, emb3, W1[:nd], w1c, b1[None, :], W2, b2[None, :],
                W3, b3[None, :], wab, bab, bm=2048)
  return out[:, 0], out[:, 1]
